# Initial kernel scaffold; baseline (speedup 1.0000x reference)
import jax
import jax.numpy as jnp
from jax.experimental import pallas as pl


def kernel(x, w_in_vals, b_in, w_out_vals, b_out, in_idx, out_idx):
    raise NotImplementedError("write your pallas kernel here")


# speedup vs baseline: 1.5140x; 1.5140x over previous
import jax
import jax.numpy as jnp
from jax.experimental import pallas as pl
from jax.experimental.pallas import tpu as pltpu

_B = 320000
_C = 128
_H = 128
_NNZ = 4096
_BLK = 2560


def _fused_body(in_idx_ref, in_vals_ref, b_in_ref, out_idx_ref, out_vals_ref,
                b_out_ref, x_ref, o_ref, wt_in_ref, wt_out_ref):
    @pl.when(pl.program_id(0) == 0)
    def _():
        rows_i = in_idx_ref[0:1, :]
        cols_i = in_idx_ref[1:2, :]
        vals_i = in_vals_ref[0:1, :]
        cmat = jnp.where(
            jax.lax.broadcasted_iota(jnp.int32, (_C, _NNZ), 0) == cols_i,
            vals_i, 0.0)
        rmat = jnp.where(
            jax.lax.broadcasted_iota(jnp.int32, (_H, _NNZ), 0) == rows_i,
            1.0, 0.0)
        wt_in_ref[...] = jax.lax.dot_general(
            cmat, rmat, (((1,), (1,)), ((), ())),
            preferred_element_type=jnp.float32)

        rows_o = out_idx_ref[0:1, :]
        cols_o = out_idx_ref[1:2, :]
        vals_o = out_vals_ref[0:1, :]
        hmat = jnp.where(
            jax.lax.broadcasted_iota(jnp.int32, (_H, _NNZ), 0) == cols_o,
            vals_o, 0.0)
        cmat2 = jnp.where(
            jax.lax.broadcasted_iota(jnp.int32, (_C, _NNZ), 0) == rows_o,
            1.0, 0.0)
        wt_out_ref[...] = jax.lax.dot_general(
            hmat, cmat2, (((1,), (1,)), ((), ())),
            preferred_element_type=jnp.float32)

    x = x_ref[...]
    h = jnp.dot(x, wt_in_ref[...], preferred_element_type=jnp.float32)
    h = jnp.maximum(h + b_in_ref[0:1, :], 0.0)
    o = jnp.dot(h, wt_out_ref[...], preferred_element_type=jnp.float32)
    o_ref[...] = o + b_out_ref[0:1, :] + x


def kernel(x, w_in_vals, b_in, w_out_vals, b_out, in_idx, out_idx):
    grid = (_B // _BLK,)
    out = pl.pallas_call(
        _fused_body,
        grid=grid,
        in_specs=[
            pl.BlockSpec((2, _NNZ), lambda i: (0, 0)),
            pl.BlockSpec((1, _NNZ), lambda i: (0, 0)),
            pl.BlockSpec((1, _C), lambda i: (0, 0)),
            pl.BlockSpec((2, _NNZ), lambda i: (0, 0)),
            pl.BlockSpec((1, _NNZ), lambda i: (0, 0)),
            pl.BlockSpec((1, _H), lambda i: (0, 0)),
            pl.BlockSpec((_BLK, _C), lambda i: (i, 0)),
        ],
        out_specs=pl.BlockSpec((_BLK, _C), lambda i: (i, 0)),
        out_shape=jax.ShapeDtypeStruct((_B, _C), jnp.float32),
        scratch_shapes=[
            pltpu.VMEM((_C, _H), jnp.float32),
            pltpu.VMEM((_H, _C), jnp.float32),
        ],
    )(
        in_idx,
        w_in_vals.reshape(1, _NNZ),
        b_in.reshape(1, _H),
        out_idx,
        w_out_vals.reshape(1, _NNZ),
        b_out.reshape(1, _C),
        x,
    )
    return out


# speedup vs baseline: 2.1953x; 1.4500x over previous
import jax
import jax.numpy as jnp
from jax.experimental import pallas as pl
from jax.experimental.pallas import tpu as pltpu

_B = 320000
_C = 128
_H = 128
_NNZ = 4096
_BLK = 6400


def _fused_body(in_idx_ref, in_vals_ref, b_in_ref, out_idx_ref, out_vals_ref,
                b_out_ref, x_ref, o_ref, wt_in_ref, wt_out_ref):
    @pl.when(pl.program_id(0) == 0)
    def _():
        rows_i = in_idx_ref[0:1, :]
        cols_i = in_idx_ref[1:2, :]
        vals_i = in_vals_ref[0:1, :]
        cmat = jnp.where(
            jax.lax.broadcasted_iota(jnp.int32, (_C, _NNZ), 0) == cols_i,
            vals_i, 0.0)
        rmat = jnp.where(
            jax.lax.broadcasted_iota(jnp.int32, (_H, _NNZ), 0) == rows_i,
            1.0, 0.0)
        wt_in_ref[...] = jax.lax.dot_general(
            cmat, rmat, (((1,), (1,)), ((), ())),
            preferred_element_type=jnp.float32)

        rows_o = out_idx_ref[0:1, :]
        cols_o = out_idx_ref[1:2, :]
        vals_o = out_vals_ref[0:1, :]
        hmat = jnp.where(
            jax.lax.broadcasted_iota(jnp.int32, (_H, _NNZ), 0) == cols_o,
            vals_o, 0.0)
        cmat2 = jnp.where(
            jax.lax.broadcasted_iota(jnp.int32, (_C, _NNZ), 0) == rows_o,
            1.0, 0.0)
        wt_out_ref[...] = jax.lax.dot_general(
            hmat, cmat2, (((1,), (1,)), ((), ())),
            preferred_element_type=jnp.float32)

    x = x_ref[...]
    h = jnp.dot(x, wt_in_ref[...], preferred_element_type=jnp.float32)
    h = jnp.maximum(h + b_in_ref[0:1, :], 0.0)
    o = jnp.dot(h, wt_out_ref[...], preferred_element_type=jnp.float32)
    o_ref[...] = o + b_out_ref[0:1, :] + x


def kernel(x, w_in_vals, b_in, w_out_vals, b_out, in_idx, out_idx):
    grid = (_B // _BLK,)
    out = pl.pallas_call(
        _fused_body,
        grid=grid,
        in_specs=[
            pl.BlockSpec((2, _NNZ), lambda i: (0, 0)),
            pl.BlockSpec((1, _NNZ), lambda i: (0, 0)),
            pl.BlockSpec((1, _C), lambda i: (0, 0)),
            pl.BlockSpec((2, _NNZ), lambda i: (0, 0)),
            pl.BlockSpec((1, _NNZ), lambda i: (0, 0)),
            pl.BlockSpec((1, _H), lambda i: (0, 0)),
            pl.BlockSpec((_BLK, _C), lambda i: (i, 0)),
        ],
        out_specs=pl.BlockSpec((_BLK, _C), lambda i: (i, 0)),
        out_shape=jax.ShapeDtypeStruct((_B, _C), jnp.float32),
        scratch_shapes=[
            pltpu.VMEM((_C, _H), jnp.float32),
            pltpu.VMEM((_H, _C), jnp.float32),
        ],
    )(
        in_idx,
        w_in_vals.reshape(1, _NNZ),
        b_in.reshape(1, _H),
        out_idx,
        w_out_vals.reshape(1, _NNZ),
        b_out.reshape(1, _C),
        x,
    )
    return out


# speedup vs baseline: 2.4000x; 1.0932x over previous
import jax
import jax.numpy as jnp
from jax.experimental import pallas as pl
from jax.experimental.pallas import tpu as pltpu

_B = 320000
_C = 128
_H = 128
_NNZ = 4096
_BLK = 12800


def _fused_body(in_idx_ref, in_vals_ref, b_in_ref, out_idx_ref, out_vals_ref,
                b_out_ref, x_ref, o_ref, wt_in_ref, wt_out_ref):
    @pl.when(pl.program_id(0) == 0)
    def _():
        rows_i = in_idx_ref[0:1, :]
        cols_i = in_idx_ref[1:2, :]
        vals_i = in_vals_ref[0:1, :]
        cmat = jnp.where(
            jax.lax.broadcasted_iota(jnp.int32, (_C, _NNZ), 0) == cols_i,
            vals_i, 0.0)
        rmat = jnp.where(
            jax.lax.broadcasted_iota(jnp.int32, (_H, _NNZ), 0) == rows_i,
            1.0, 0.0)
        wt_in_ref[...] = jax.lax.dot_general(
            cmat, rmat, (((1,), (1,)), ((), ())),
            preferred_element_type=jnp.float32)

        rows_o = out_idx_ref[0:1, :]
        cols_o = out_idx_ref[1:2, :]
        vals_o = out_vals_ref[0:1, :]
        hmat = jnp.where(
            jax.lax.broadcasted_iota(jnp.int32, (_H, _NNZ), 0) == cols_o,
            vals_o, 0.0)
        cmat2 = jnp.where(
            jax.lax.broadcasted_iota(jnp.int32, (_C, _NNZ), 0) == rows_o,
            1.0, 0.0)
        wt_out_ref[...] = jax.lax.dot_general(
            hmat, cmat2, (((1,), (1,)), ((), ())),
            preferred_element_type=jnp.float32)

    x = x_ref[...]
    h = jnp.dot(x, wt_in_ref[...], preferred_element_type=jnp.float32)
    h = jnp.maximum(h + b_in_ref[0:1, :], 0.0)
    o = jnp.dot(h, wt_out_ref[...], preferred_element_type=jnp.float32)
    o_ref[...] = o + b_out_ref[0:1, :] + x


def kernel(x, w_in_vals, b_in, w_out_vals, b_out, in_idx, out_idx):
    grid = (_B // _BLK,)
    out = pl.pallas_call(
        _fused_body,
        grid=grid,
        in_specs=[
            pl.BlockSpec((2, _NNZ), lambda i: (0, 0)),
            pl.BlockSpec((1, _NNZ), lambda i: (0, 0)),
            pl.BlockSpec((1, _C), lambda i: (0, 0)),
            pl.BlockSpec((2, _NNZ), lambda i: (0, 0)),
            pl.BlockSpec((1, _NNZ), lambda i: (0, 0)),
            pl.BlockSpec((1, _H), lambda i: (0, 0)),
            pl.BlockSpec((_BLK, _C), lambda i: (i, 0)),
        ],
        out_specs=pl.BlockSpec((_BLK, _C), lambda i: (i, 0)),
        out_shape=jax.ShapeDtypeStruct((_B, _C), jnp.float32),
        scratch_shapes=[
            pltpu.VMEM((_C, _H), jnp.float32),
            pltpu.VMEM((_H, _C), jnp.float32),
        ],
    )(
        in_idx,
        w_in_vals.reshape(1, _NNZ),
        b_in.reshape(1, _H),
        out_idx,
        w_out_vals.reshape(1, _NNZ),
        b_out.reshape(1, _C),
        x,
    )
    return out


# speedup vs baseline: 2.4088x; 1.0037x over previous
import jax
import jax.numpy as jnp
from jax.experimental import pallas as pl
from jax.experimental.pallas import tpu as pltpu

_B = 320000
_C = 128
_H = 128
_NNZ = 4096
_BLK = 16000


def _fused_body(in_idx_ref, in_vals_ref, b_in_ref, out_idx_ref, out_vals_ref,
                b_out_ref, x_ref, o_ref, wt_in_ref, wt_out_ref):
    @pl.when(pl.program_id(0) == 0)
    def _():
        rows_i = in_idx_ref[0:1, :]
        cols_i = in_idx_ref[1:2, :]
        vals_i = in_vals_ref[0:1, :]
        cmat = jnp.where(
            jax.lax.broadcasted_iota(jnp.int32, (_C, _NNZ), 0) == cols_i,
            vals_i, 0.0)
        rmat = jnp.where(
            jax.lax.broadcasted_iota(jnp.int32, (_H, _NNZ), 0) == rows_i,
            1.0, 0.0)
        wt_in_ref[...] = jax.lax.dot_general(
            cmat, rmat, (((1,), (1,)), ((), ())),
            preferred_element_type=jnp.float32)

        rows_o = out_idx_ref[0:1, :]
        cols_o = out_idx_ref[1:2, :]
        vals_o = out_vals_ref[0:1, :]
        hmat = jnp.where(
            jax.lax.broadcasted_iota(jnp.int32, (_H, _NNZ), 0) == cols_o,
            vals_o, 0.0)
        cmat2 = jnp.where(
            jax.lax.broadcasted_iota(jnp.int32, (_C, _NNZ), 0) == rows_o,
            1.0, 0.0)
        wt_out_ref[...] = jax.lax.dot_general(
            hmat, cmat2, (((1,), (1,)), ((), ())),
            preferred_element_type=jnp.float32)

    x = x_ref[...]
    h = jnp.dot(x, wt_in_ref[...], preferred_element_type=jnp.float32)
    h = jnp.maximum(h + b_in_ref[0:1, :], 0.0)
    o = jnp.dot(h, wt_out_ref[...], preferred_element_type=jnp.float32)
    o_ref[...] = o + b_out_ref[0:1, :] + x


def kernel(x, w_in_vals, b_in, w_out_vals, b_out, in_idx, out_idx):
    grid = (_B // _BLK,)
    out = pl.pallas_call(
        _fused_body,
        grid=grid,
        in_specs=[
            pl.BlockSpec((2, _NNZ), lambda i: (0, 0)),
            pl.BlockSpec((1, _NNZ), lambda i: (0, 0)),
            pl.BlockSpec((1, _C), lambda i: (0, 0)),
            pl.BlockSpec((2, _NNZ), lambda i: (0, 0)),
            pl.BlockSpec((1, _NNZ), lambda i: (0, 0)),
            pl.BlockSpec((1, _H), lambda i: (0, 0)),
            pl.BlockSpec((_BLK, _C), lambda i: (i, 0)),
        ],
        out_specs=pl.BlockSpec((_BLK, _C), lambda i: (i, 0)),
        out_shape=jax.ShapeDtypeStruct((_B, _C), jnp.float32),
        scratch_shapes=[
            pltpu.VMEM((_C, _H), jnp.float32),
            pltpu.VMEM((_H, _C), jnp.float32),
        ],
    )(
        in_idx,
        w_in_vals.reshape(1, _NNZ),
        b_in.reshape(1, _H),
        out_idx,
        w_out_vals.reshape(1, _NNZ),
        b_out.reshape(1, _C),
        x,
    )
    return out
